# Initial kernel scaffold; baseline (speedup 1.0000x reference)
import jax
import jax.numpy as jnp
from jax.experimental import pallas as pl


def kernel(h, r, t, neg_idx, entity_table, relation_table):
    raise NotImplementedError("write your pallas kernel here")


# speedup vs baseline: 1.0767x; 1.0767x over previous
import jax
import jax.numpy as jnp
from jax import lax
from jax.experimental import pallas as pl
from jax.experimental.pallas import tpu as pltpu, tpu_sc as plsc

B = 16384
D = 64
NC, NS, L = 2, 16, 16
NW = NC * NS
RPW = B // NW
C = 256
NCHUNK = RPW // C


def _body(h_hbm, r_hbm, t_hbm, n_hbm, ent_hbm, rel_hbm, score_hbm, neg_hbm,
          hi, ri, ti, ni, hb, rb, tb, nb, sem_h, sem_r, sem_t, sem_n):
    wid = lax.axis_index("s") * NC + lax.axis_index("c")
    base_w = wid * RPW
    for chunk in range(NCHUNK):
        base = base_w + chunk * C
        pltpu.sync_copy(h_hbm.at[pl.ds(base, C)], hi)
        pltpu.sync_copy(r_hbm.at[pl.ds(base, C)], ri)
        pltpu.sync_copy(t_hbm.at[pl.ds(base, C)], ti)
        pltpu.sync_copy(n_hbm.at[pl.ds(base, C)], ni)
        ch = pltpu.async_copy(ent_hbm.at[hi], hb, sem_h)
        cr = pltpu.async_copy(rel_hbm.at[ri], rb, sem_r)
        ct = pltpu.async_copy(ent_hbm.at[ti], tb, sem_t)
        cn = pltpu.async_copy(ent_hbm.at[ni], nb, sem_n)
        ch.wait(); cr.wait(); ct.wait(); cn.wait()

        @pl.loop(0, C)
        def _compute(i):
            for j in range(D // L):
                sl = pl.ds(j * L, L)
                s = hb[i, sl] + rb[i, sl]
                tb[i, sl] = s - tb[i, sl]
                nb[i, sl] = s - nb[i, sl]

        pltpu.sync_copy(tb, score_hbm.at[pl.ds(base, C)])
        pltpu.sync_copy(nb, neg_hbm.at[pl.ds(base, C)])


def kernel(h, r, t, neg_idx, entity_table, relation_table):
    mesh = plsc.VectorSubcoreMesh(
        core_axis_name="c", subcore_axis_name="s",
        num_cores=NC, num_subcores=NS)
    f = pl.kernel(
        _body,
        out_type=(jax.ShapeDtypeStruct((B, D), jnp.float32),
                  jax.ShapeDtypeStruct((B, D), jnp.float32)),
        mesh=mesh,
        compiler_params=pltpu.CompilerParams(use_tc_tiling_on_sc=False),
        scratch_types=[
            pltpu.VMEM((C,), jnp.int32),
            pltpu.VMEM((C,), jnp.int32),
            pltpu.VMEM((C,), jnp.int32),
            pltpu.VMEM((C,), jnp.int32),
            pltpu.VMEM((C, D), jnp.float32),
            pltpu.VMEM((C, D), jnp.float32),
            pltpu.VMEM((C, D), jnp.float32),
            pltpu.VMEM((C, D), jnp.float32),
            pltpu.SemaphoreType.DMA,
            pltpu.SemaphoreType.DMA,
            pltpu.SemaphoreType.DMA,
            pltpu.SemaphoreType.DMA,
        ],
    )
    score, neg = f(h.astype(jnp.int32), r.astype(jnp.int32),
                   t.astype(jnp.int32), neg_idx.astype(jnp.int32),
                   entity_table, relation_table)
    return score[:, None, :], neg[:, None, :]


# speedup vs baseline: 1.1188x; 1.0391x over previous
import jax
import jax.numpy as jnp
from jax import lax
from jax.experimental import pallas as pl
from jax.experimental.pallas import tpu as pltpu, tpu_sc as plsc

B = 16384
D = 64
NC, NS, L = 2, 16, 16
NW = NC * NS
RPW = B // NW
C = 128
NCHUNK = RPW // C


def _body(h_hbm, r_hbm, t_hbm, n_hbm, ent_hbm, rel_hbm, score_hbm, neg_hbm,
          hi, ri, ti, ni,
          hb0, rb0, tb0, nb0, hb1, rb1, tb1, nb1,
          gsem0, gsem1, osem0, osem1, isem):
    wid = lax.axis_index("s") * NC + lax.axis_index("c")
    base_w = wid * RPW

    idx_copies = []
    for k in range(NCHUNK):
        off = base_w + k * C
        idx_copies.append(pltpu.async_copy(h_hbm.at[pl.ds(off, C)], hi.at[k], isem))
        idx_copies.append(pltpu.async_copy(r_hbm.at[pl.ds(off, C)], ri.at[k], isem))
        idx_copies.append(pltpu.async_copy(t_hbm.at[pl.ds(off, C)], ti.at[k], isem))
        idx_copies.append(pltpu.async_copy(n_hbm.at[pl.ds(off, C)], ni.at[k], isem))
    for c in idx_copies:
        c.wait()

    sets = ((hb0, rb0, tb0, nb0, gsem0, osem0),
            (hb1, rb1, tb1, nb1, gsem1, osem1))

    def start_gathers(k):
        hb, rb, tb, nb, gsem, _ = sets[k % 2]
        return [pltpu.async_copy(ent_hbm.at[hi.at[k]], hb, gsem),
                pltpu.async_copy(rel_hbm.at[ri.at[k]], rb, gsem),
                pltpu.async_copy(ent_hbm.at[ti.at[k]], tb, gsem),
                pltpu.async_copy(ent_hbm.at[ni.at[k]], nb, gsem)]

    pend_g = {0: start_gathers(0)}
    pend_o = {}
    for k in range(NCHUNK):
        hb, rb, tb, nb, gsem, osem = sets[k % 2]
        if k + 1 < NCHUNK:
            for c in pend_o.pop(k - 1, ()):
                c.wait()
            pend_g[k + 1] = start_gathers(k + 1)
        for c in pend_g.pop(k):
            c.wait()

        @plsc.parallel_loop(0, C)
        def _compute(i):
            for j in range(D // L):
                sl = pl.ds(j * L, L)
                s = hb[i, sl] + rb[i, sl]
                tb[i, sl] = s - tb[i, sl]
                nb[i, sl] = s - nb[i, sl]

        off = base_w + k * C
        pend_o[k] = [pltpu.async_copy(tb, score_hbm.at[pl.ds(off, C)], osem),
                     pltpu.async_copy(nb, neg_hbm.at[pl.ds(off, C)], osem)]
    for k in sorted(pend_o):
        for c in pend_o[k]:
            c.wait()


def kernel(h, r, t, neg_idx, entity_table, relation_table):
    mesh = plsc.VectorSubcoreMesh(
        core_axis_name="c", subcore_axis_name="s",
        num_cores=NC, num_subcores=NS)
    f = pl.kernel(
        _body,
        out_type=(jax.ShapeDtypeStruct((B, D), jnp.float32),
                  jax.ShapeDtypeStruct((B, D), jnp.float32)),
        mesh=mesh,
        compiler_params=pltpu.CompilerParams(use_tc_tiling_on_sc=False),
        scratch_types=[
            pltpu.VMEM((NCHUNK, C), jnp.int32),
            pltpu.VMEM((NCHUNK, C), jnp.int32),
            pltpu.VMEM((NCHUNK, C), jnp.int32),
            pltpu.VMEM((NCHUNK, C), jnp.int32),
            pltpu.VMEM((C, D), jnp.float32),
            pltpu.VMEM((C, D), jnp.float32),
            pltpu.VMEM((C, D), jnp.float32),
            pltpu.VMEM((C, D), jnp.float32),
            pltpu.VMEM((C, D), jnp.float32),
            pltpu.VMEM((C, D), jnp.float32),
            pltpu.VMEM((C, D), jnp.float32),
            pltpu.VMEM((C, D), jnp.float32),
            pltpu.SemaphoreType.DMA,
            pltpu.SemaphoreType.DMA,
            pltpu.SemaphoreType.DMA,
            pltpu.SemaphoreType.DMA,
            pltpu.SemaphoreType.DMA,
        ],
    )
    score, neg = f(h.astype(jnp.int32), r.astype(jnp.int32),
                   t.astype(jnp.int32), neg_idx.astype(jnp.int32),
                   entity_table, relation_table)
    return score[:, None, :], neg[:, None, :]
